# baseline (speedup 1.0000x reference)
import functools

import jax
import jax.numpy as jnp
from jax import lax
from jax.experimental import pallas as pl
from jax.experimental.pallas import tpu as pltpu
from jax.experimental.pallas import tpu_sc as plsc

DIM = 128
LANES = 16
NUM_CORES = 2
NUM_SUBCORES = 16
NUM_WORKERS = NUM_CORES * NUM_SUBCORES
NLG = DIM // LANES

COFF = (0, 128)
CLEN = (128, 72)


def _build_kernel(B, N, S, V):
    assert S == 2 and DIM == 128
    assert B % NUM_WORKERS == 0
    assert COFF[1] + CLEN[1] == N and all(c % 8 == 0 for c in COFF)
    b_per_w = B // NUM_WORKERS

    mesh = plsc.VectorSubcoreMesh(core_axis_name="c", subcore_axis_name="s")

    ob_shapes = [pltpu.VMEM((2 * CLEN[s % 2], DIM), jnp.float32)
                 for s in range(4)]
    ix_shapes = [pltpu.VMEM((CLEN[s % 2],), jnp.int32) for s in range(4)]

    @functools.partial(
        pl.kernel,
        mesh=mesh,
        out_type=jax.ShapeDtypeStruct((B * S * N, DIM), jnp.float32),
        scratch_types=ob_shapes + ix_shapes + [
            pltpu.VMEM((N, DIM), jnp.float32),
            pltpu.VMEM((S, DIM), jnp.float32),
        ] + [pltpu.SemaphoreType.DMA] * 13,
    )
    def k(x_hbm, table_hbm, pos_hbm, stream_hbm, out_hbm,
          ob0, ob1, ob2, ob3, ix0, ix1, ix2, ix3, pos_v, stream_v,
          gsem0, gsem1, gsem2, gsem3, wsem0, wsem1, wsem2, wsem3,
          isem0, isem1, isem2, isem3, psem):
        ob = (ob0, ob1, ob2, ob3)
        ix = (ix0, ix1, ix2, ix3)
        gsem = (gsem0, gsem1, gsem2, gsem3)
        wsem = (wsem0, wsem1, wsem2, wsem3)
        isem = (isem0, isem1, isem2, isem3)

        wid = lax.axis_index("s") * NUM_CORES + lax.axis_index("c")

        def b_of(q):
            return q * NUM_WORKERS + wid

        pltpu.async_copy(pos_hbm.at[pl.ds(0, N)], pos_v, psem)
        pltpu.async_copy(stream_hbm, stream_v, psem)

        def idx_fetch(q, h, s):
            boff = jnp.minimum(b_of(q), B - 1) * N + COFF[h]
            pltpu.async_copy(
                x_hbm.at[pl.ds(boff, CLEN[h])], ix[s], isem[s])

        def idx_wait(s):
            pltpu.make_async_copy(
                x_hbm.at[pl.ds(0, CLEN[s % 2])], ix[s], isem[s]).wait()

        def drain_w(s):
            h = s % 2
            pltpu.make_async_copy(
                ob[s].at[pl.ds(0, CLEN[h])],
                out_hbm.at[pl.ds(0, CLEN[h])], wsem[s]).wait()
            pltpu.make_async_copy(
                ob[s].at[pl.ds(CLEN[h], CLEN[h])],
                out_hbm.at[pl.ds(0, CLEN[h])], wsem[s]).wait()

        def a_stage(q, h, s, first):
            if not first:
                drain_w(s)
            idx_wait(s)
            pltpu.async_copy(
                table_hbm.at[ix[s]], ob[s].at[pl.ds(0, CLEN[h])], gsem[s])

        def b_stage(q, h, s):
            clen = CLEN[h]
            pltpu.make_async_copy(
                table_hbm.at[ix[s]], ob[s].at[pl.ds(0, clen)], gsem[s]).wait()
            idx_fetch(q + 2, h, s)

            def body_n(n, carry_n):
                for l in range(NLG):
                    sl = pl.ds(l * LANES, LANES)
                    t0 = ob[s][n, sl] + pos_v[COFF[h] + n, sl] + s0[l]
                    ob[s][n, sl] = t0
                    ob[s][clen + n, sl] = t0 + d[l]
                return carry_n

            lax.fori_loop(0, clen, body_n, 0)

            wrow = b_of(q) * (S * N) + COFF[h]
            pltpu.async_copy(
                ob[s].at[pl.ds(0, clen)],
                out_hbm.at[pl.ds(pl.multiple_of(wrow, 8), clen)], wsem[s])
            pltpu.async_copy(
                ob[s].at[pl.ds(clen, clen)],
                out_hbm.at[pl.ds(pl.multiple_of(wrow + N, 8), clen)], wsem[s])

        for s in range(4):
            idx_fetch(s // 2, s % 2, s)

        pltpu.make_async_copy(pos_hbm.at[pl.ds(0, N)], pos_v, psem).wait()
        pltpu.make_async_copy(stream_hbm, stream_v, psem).wait()
        s0 = [stream_v[0, pl.ds(l * LANES, LANES)] for l in range(NLG)]
        d = [stream_v[1, pl.ds(l * LANES, LANES)] - s0[l] for l in range(NLG)]

        def body_i(i, carry):
            for p in range(4):
                q = 2 * i + p // 2
                h = p % 2

                @pl.when(i > 0)
                def _():
                    a_stage(q, h, p, first=False)

                @pl.when(i == 0)
                def _():
                    a_stage(q, h, p, first=True)

                pq = 2 * i + (p - 1) // 2 if p > 0 else 2 * i - 1
                ph = (p - 1) % 2
                ps = (p - 1) % 4
                if p > 0:
                    b_stage(pq, ph, ps)
                else:
                    @pl.when(i > 0)
                    def _():
                        b_stage(pq, ph, ps)
            return carry

        lax.fori_loop(0, b_per_w // 2, body_i, 0)

        b_stage(b_per_w - 1, 1, 3)
        for s in range(4):
            drain_w(s)
            idx_wait(s)

    return k


def kernel(x, token_table, pos_table, stream_emb):
    B, N = x.shape
    S, D = stream_emb.shape
    V = token_table.shape[0]
    xflat = x.reshape(B * N).astype(jnp.int32)
    k = _build_kernel(B, N, S, V)
    out = k(xflat, token_table, pos_table, stream_emb)
    return out.reshape(B * S, N, D)


# speedup vs baseline: 1.0050x; 1.0050x over previous
import functools

import jax
import jax.numpy as jnp
from jax import lax
from jax.experimental import pallas as pl
from jax.experimental.pallas import tpu as pltpu
from jax.experimental.pallas import tpu_sc as plsc

DIM = 128
LANES = 16
NUM_CORES = 2
NUM_SUBCORES = 16
NUM_WORKERS = NUM_CORES * NUM_SUBCORES
NLG = DIM // LANES

COFF = (0, 104)
CLEN = (104, 96)


def _build_kernel(B, N, S, V):
    assert S == 2 and DIM == 128
    assert B % NUM_WORKERS == 0
    assert COFF[1] + CLEN[1] == N and all(c % 8 == 0 for c in COFF)
    b_per_w = B // NUM_WORKERS

    mesh = plsc.VectorSubcoreMesh(core_axis_name="c", subcore_axis_name="s")

    ob_shapes = [pltpu.VMEM((2 * CLEN[s % 2], DIM), jnp.float32)
                 for s in range(4)]
    ix_shapes = [pltpu.VMEM((CLEN[s % 2],), jnp.int32) for s in range(4)]

    @functools.partial(
        pl.kernel,
        mesh=mesh,
        out_type=jax.ShapeDtypeStruct((B * S * N, DIM), jnp.float32),
        scratch_types=ob_shapes + ix_shapes + [
            pltpu.VMEM((N, DIM), jnp.float32),
            pltpu.VMEM((S, DIM), jnp.float32),
        ] + [pltpu.SemaphoreType.DMA] * 13,
    )
    def k(x_hbm, table_hbm, pos_hbm, stream_hbm, out_hbm,
          ob0, ob1, ob2, ob3, ix0, ix1, ix2, ix3, pos_v, stream_v,
          gsem0, gsem1, gsem2, gsem3, wsem0, wsem1, wsem2, wsem3,
          isem0, isem1, isem2, isem3, psem):
        ob = (ob0, ob1, ob2, ob3)
        ix = (ix0, ix1, ix2, ix3)
        gsem = (gsem0, gsem1, gsem2, gsem3)
        wsem = (wsem0, wsem1, wsem2, wsem3)
        isem = (isem0, isem1, isem2, isem3)

        wid = lax.axis_index("s") * NUM_CORES + lax.axis_index("c")

        def b_of(q):
            return q * NUM_WORKERS + wid

        pltpu.async_copy(pos_hbm.at[pl.ds(0, N)], pos_v, psem)
        pltpu.async_copy(stream_hbm, stream_v, psem)

        def idx_fetch(q, h, s):
            boff = jnp.minimum(b_of(q), B - 1) * N + COFF[h]
            pltpu.async_copy(
                x_hbm.at[pl.ds(boff, CLEN[h])], ix[s], isem[s])

        def idx_wait(s):
            pltpu.make_async_copy(
                x_hbm.at[pl.ds(0, CLEN[s % 2])], ix[s], isem[s]).wait()

        def drain_w(s):
            h = s % 2
            pltpu.make_async_copy(
                ob[s].at[pl.ds(0, CLEN[h])],
                out_hbm.at[pl.ds(0, CLEN[h])], wsem[s]).wait()
            pltpu.make_async_copy(
                ob[s].at[pl.ds(CLEN[h], CLEN[h])],
                out_hbm.at[pl.ds(0, CLEN[h])], wsem[s]).wait()

        def a_stage(q, h, s, first):
            if not first:
                drain_w(s)
            idx_wait(s)
            pltpu.async_copy(
                table_hbm.at[ix[s]], ob[s].at[pl.ds(0, CLEN[h])], gsem[s])

        def b_stage(q, h, s):
            clen = CLEN[h]
            pltpu.make_async_copy(
                table_hbm.at[ix[s]], ob[s].at[pl.ds(0, clen)], gsem[s]).wait()
            idx_fetch(q + 2, h, s)

            def body_n(n, carry_n):
                for l in range(NLG):
                    sl = pl.ds(l * LANES, LANES)
                    t0 = ob[s][n, sl] + pos_v[COFF[h] + n, sl] + s0[l]
                    ob[s][n, sl] = t0
                    ob[s][clen + n, sl] = t0 + d[l]
                return carry_n

            lax.fori_loop(0, clen, body_n, 0)

            wrow = b_of(q) * (S * N) + COFF[h]
            pltpu.async_copy(
                ob[s].at[pl.ds(0, clen)],
                out_hbm.at[pl.ds(pl.multiple_of(wrow, 8), clen)], wsem[s])
            pltpu.async_copy(
                ob[s].at[pl.ds(clen, clen)],
                out_hbm.at[pl.ds(pl.multiple_of(wrow + N, 8), clen)], wsem[s])

        for s in range(4):
            idx_fetch(s // 2, s % 2, s)

        pltpu.make_async_copy(pos_hbm.at[pl.ds(0, N)], pos_v, psem).wait()
        pltpu.make_async_copy(stream_hbm, stream_v, psem).wait()
        s0 = [stream_v[0, pl.ds(l * LANES, LANES)] for l in range(NLG)]
        d = [stream_v[1, pl.ds(l * LANES, LANES)] - s0[l] for l in range(NLG)]

        def body_i(i, carry):
            for p in range(4):
                q = 2 * i + p // 2
                h = p % 2

                @pl.when(i > 0)
                def _():
                    a_stage(q, h, p, first=False)

                @pl.when(i == 0)
                def _():
                    a_stage(q, h, p, first=True)

                pq = 2 * i + (p - 1) // 2 if p > 0 else 2 * i - 1
                ph = (p - 1) % 2
                ps = (p - 1) % 4
                if p > 0:
                    b_stage(pq, ph, ps)
                else:
                    @pl.when(i > 0)
                    def _():
                        b_stage(pq, ph, ps)
            return carry

        lax.fori_loop(0, b_per_w // 2, body_i, 0)

        b_stage(b_per_w - 1, 1, 3)
        for s in range(4):
            drain_w(s)
            idx_wait(s)

    return k


def kernel(x, token_table, pos_table, stream_emb):
    B, N = x.shape
    S, D = stream_emb.shape
    V = token_table.shape[0]
    xflat = x.reshape(B * N).astype(jnp.int32)
    k = _build_kernel(B, N, S, V)
    out = k(xflat, token_table, pos_table, stream_emb)
    return out.reshape(B * S, N, D)
